# baseline (speedup 1.0000x reference)
import functools
import math

import jax
import jax.numpy as jnp
from jax import lax
from jax.experimental import pallas as pl
from jax.experimental.pallas import tpu as pltpu
from jax.experimental.pallas import tpu_sc as plsc

_N = 10000
_E = 320000
_D_IN = 128
_HID = 128
_D_OUT = 64
_NE = 8
_COEF = 0.01
_NUM_LAYERS = 4

_NP = 10240
_ROWS_PER_TILE = _NP // 16
_BA = 1000
_GRID = _N // _BA
_EDGE_B = 128
_NW = 32
_K = 79
_EPAD = _NW * _K * _EDGE_B


def _prelude_body(x_ref, noise_ref, win_ref, bin_ref, wg_ref, wn_ref,
                  h_ref, logits_ref, gates_ref, imp_ref, load_ref):
    i = pl.program_id(0)
    x = x_ref[...]
    h = jnp.maximum(
        jnp.dot(x, win_ref[...], preferred_element_type=jnp.float32)
        + bin_ref[...], 0.0)
    h_ref[...] = h
    clean = jnp.dot(h, wg_ref[...], preferred_element_type=jnp.float32)
    noisy = jnp.dot(h, wn_ref[...], preferred_element_type=jnp.float32)
    sp = jnp.maximum(noisy, 0.0) + jnp.log1p(jnp.exp(-jnp.abs(noisy)))
    raw = clean + noise_ref[...] * (sp + 1e-2)
    logits_ref[...] = raw
    col = lax.broadcasted_iota(jnp.int32, raw.shape, 1)
    m1 = jnp.max(raw, axis=1, keepdims=True)
    i1 = jnp.min(jnp.where(raw == m1, col, _NE), axis=1, keepdims=True)
    masked = jnp.where(col == i1, -jnp.inf, raw)
    m2 = jnp.max(masked, axis=1, keepdims=True)
    i2 = jnp.min(jnp.where(masked == m2, col, _NE), axis=1, keepdims=True)
    e2 = jnp.exp(m2 - m1)
    denom = 1.0 + e2
    gates = (jnp.where(col == i1, 1.0 / denom, 0.0)
             + jnp.where(col == i2, e2 / denom, 0.0))
    gates_ref[...] = gates

    @pl.when(i == 0)
    def _():
        imp_ref[...] = jnp.zeros_like(imp_ref)
        load_ref[...] = jnp.zeros_like(load_ref)

    imp_ref[...] += jnp.sum(gates, axis=0, keepdims=True)
    load_ref[...] += jnp.sum(
        jnp.where(gates > 0.0, 1.0, 0.0), axis=0, keepdims=True)


def _prelude(x, noise, w_in, b_in2, w_gate, w_noise):
    return pl.pallas_call(
        _prelude_body,
        grid=(_GRID,),
        in_specs=[
            pl.BlockSpec((_BA, _D_IN), lambda i: (i, 0)),
            pl.BlockSpec((_BA, _NE), lambda i: (i, 0)),
            pl.BlockSpec((_D_IN, _HID), lambda i: (0, 0)),
            pl.BlockSpec((1, _HID), lambda i: (0, 0)),
            pl.BlockSpec((_HID, _NE), lambda i: (0, 0)),
            pl.BlockSpec((_HID, _NE), lambda i: (0, 0)),
        ],
        out_specs=[
            pl.BlockSpec((_BA, _HID), lambda i: (i, 0)),
            pl.BlockSpec((_BA, _NE), lambda i: (i, 0)),
            pl.BlockSpec((_BA, _NE), lambda i: (i, 0)),
            pl.BlockSpec((1, _NE), lambda i: (0, 0)),
            pl.BlockSpec((1, _NE), lambda i: (0, 0)),
        ],
        out_shape=[
            jax.ShapeDtypeStruct((_N, _HID), jnp.float32),
            jax.ShapeDtypeStruct((_N, _NE), jnp.float32),
            jax.ShapeDtypeStruct((_N, _NE), jnp.float32),
            jax.ShapeDtypeStruct((1, _NE), jnp.float32),
            jax.ShapeDtypeStruct((1, _NE), jnp.float32),
        ],
    )(x, noise, w_in, b_in2, w_gate, w_noise)


@functools.cache
def _make_sc_aggregate():
    mesh = plsc.VectorSubcoreMesh(core_axis_name="c", subcore_axis_name="s")

    @functools.partial(
        pl.kernel,
        mesh=mesh,
        out_type=[
            jax.ShapeDtypeStruct((2, _NP, _HID), jnp.float32),
            jax.ShapeDtypeStruct((2, _NP), jnp.float32),
        ],
        scratch_types=[
            pltpu.VMEM((_EDGE_B, _HID), jnp.float32),
            pltpu.VMEM((_K, _EDGE_B), jnp.int32),
            pltpu.VMEM((_K, _EDGE_B), jnp.int32),
            pltpu.VMEM((_EDGE_B,), jnp.float32),
            pltpu.VMEM((_ROWS_PER_TILE,), jnp.float32),
            pltpu.VMEM_SHARED((_NP, _HID), jnp.float32),
            pltpu.VMEM_SHARED((_NP,), jnp.float32),
            pltpu.SemaphoreType.DMA,
        ],
    )
    def sc_aggregate(src_hbm, dst_hbm, h_hbm, agg_out, deg_out,
                     rows_a, src_v, dst_v, ones_v, dz_v,
                     agg_sh, deg_sh, g_a):
        c = lax.axis_index("c")
        s = lax.axis_index("s")
        wid = c * 16 + s
        base = s * _ROWS_PER_TILE

        def _zrow(r, _):
            def _zlane(j, _):
                rows_a[r, pl.ds(j * 16, 16)] = jnp.zeros((16,), jnp.float32)
                return 0
            return lax.fori_loop(0, _HID // 16, _zlane, 0)
        lax.fori_loop(0, _EDGE_B, _zrow, 0)

        def _fill(j, _):
            ones_v[pl.ds(j * 16, 16)] = jnp.full((16,), 1.0, jnp.float32)
            return 0
        lax.fori_loop(0, _EDGE_B // 16, _fill, 0)

        def _fill2(j, _):
            dz_v[pl.ds(j * 16, 16)] = jnp.zeros((16,), jnp.float32)
            return 0
        lax.fori_loop(0, _ROWS_PER_TILE // 16, _fill2, 0)

        for t in range(_ROWS_PER_TILE // _EDGE_B):
            pltpu.sync_copy(rows_a, agg_sh.at[pl.ds(base + t * _EDGE_B,
                                                    _EDGE_B)])
        pltpu.sync_copy(dz_v, deg_sh.at[pl.ds(base, _ROWS_PER_TILE)])
        plsc.subcore_barrier()

        pltpu.sync_copy(src_hbm.at[wid], src_v)
        pltpu.sync_copy(dst_hbm.at[wid], dst_v)
        pltpu.async_copy(h_hbm.at[src_v.at[0]], rows_a, g_a)

        def _edge_row(j, _):
            pltpu.make_async_copy(h_hbm.at[src_v.at[j]], rows_a, g_a).wait()
            pltpu.sync_copy(rows_a, agg_sh.at[dst_v.at[j]], add=True)

            @pl.when(j < _K - 1)
            def _():
                pltpu.async_copy(h_hbm.at[src_v.at[j + 1]], rows_a, g_a)

            pltpu.sync_copy(ones_v, deg_sh.at[dst_v.at[j]], add=True)
            return 0
        lax.fori_loop(0, _K, _edge_row, 0)
        plsc.subcore_barrier()

        pltpu.sync_copy(agg_sh.at[pl.ds(base, _ROWS_PER_TILE)],
                        agg_out.at[c, pl.ds(base, _ROWS_PER_TILE)])
        pltpu.sync_copy(deg_sh.at[pl.ds(base, _ROWS_PER_TILE)],
                        deg_out.at[c, pl.ds(base, _ROWS_PER_TILE)])

    return sc_aggregate


def _expert_body(aggp_ref, degp_ref, gates_ref, we_ref, be_ref, wo_ref,
                 bo_ref, imp_ref, load_ref, out_ref, lb_ref):
    i = pl.program_id(0)
    agg = aggp_ref[0] + aggp_ref[1]
    deg = degp_ref[0] + degp_ref[1]
    agg = agg / jnp.maximum(deg, 1.0)
    acc = jnp.zeros((_BA, _HID), jnp.float32)
    g = gates_ref[...]
    for e in range(_NE):
        eo = jnp.maximum(
            jnp.dot(agg, we_ref[e], preferred_element_type=jnp.float32)
            + be_ref[...][e:e + 1, :], 0.0)
        acc += g[:, e:e + 1] * eo
    out_ref[...] = (jnp.dot(acc, wo_ref[...],
                            preferred_element_type=jnp.float32)
                    + bo_ref[...])

    @pl.when(i == 0)
    def _():
        def cv2(v):
            m = jnp.sum(v) / _NE
            var = jnp.sum((v - m) ** 2) / (_NE - 1)
            return var / (m * m + 1e-10)
        lb = _COEF * (cv2(imp_ref[...]) + cv2(load_ref[...]))
        lb = lb / math.ceil((_NUM_LAYERS - 2) / 2)
        lb_ref[...] = jnp.full((1, 1), lb, jnp.float32)


def _experts(aggp, degp_r, gates, w_expert, b_expert, w_out, b_out2,
             imp, load):
    return pl.pallas_call(
        _expert_body,
        grid=(_GRID,),
        in_specs=[
            pl.BlockSpec((2, _BA, _HID), lambda i: (0, i, 0)),
            pl.BlockSpec((2, _BA, 1), lambda i: (0, i, 0)),
            pl.BlockSpec((_BA, _NE), lambda i: (i, 0)),
            pl.BlockSpec((_NE, _HID, _HID), lambda i: (0, 0, 0)),
            pl.BlockSpec((_NE, _HID), lambda i: (0, 0)),
            pl.BlockSpec((_HID, _D_OUT), lambda i: (0, 0)),
            pl.BlockSpec((1, _D_OUT), lambda i: (0, 0)),
            pl.BlockSpec((1, _NE), lambda i: (0, 0)),
            pl.BlockSpec((1, _NE), lambda i: (0, 0)),
        ],
        out_specs=[
            pl.BlockSpec((_BA, _D_OUT), lambda i: (i, 0)),
            pl.BlockSpec((1, 1), lambda i: (0, 0)),
        ],
        out_shape=[
            jax.ShapeDtypeStruct((_N, _D_OUT), jnp.float32),
            jax.ShapeDtypeStruct((1, 1), jnp.float32),
        ],
    )(aggp, degp_r, gates, w_expert, b_expert, w_out, b_out2, imp, load)


def kernel(x, edge_index, noise, W_in, b_in, w_gate, w_noise, W_expert,
           b_expert, W_out, b_out):
    h, raw, gates, imp, load = _prelude(
        x, noise, W_in, b_in.reshape(1, _HID), w_gate, w_noise)

    src = edge_index[0]
    dst = edge_index[1]
    npad = _EPAD - _E
    src_p = jnp.concatenate(
        [src, jnp.zeros((npad,), jnp.int32)]).reshape(_NW, _K, _EDGE_B)
    pad_dst = _N + (jnp.arange(npad, dtype=jnp.int32) % (_NP - _N))
    dst_p = jnp.concatenate([dst, pad_dst]).reshape(_NW, _K, _EDGE_B)

    aggp, degp = _make_sc_aggregate()(src_p, dst_p, h)

    out, lb = _experts(aggp, degp.reshape(2, _NP, 1), gates, W_expert,
                       b_expert, W_out, b_out.reshape(1, _D_OUT),
                       imp, load)
    return out, lb.reshape(()), raw


# speedup vs baseline: 1.0010x; 1.0010x over previous
import functools
import math

import jax
import jax.numpy as jnp
from jax import lax
from jax.experimental import pallas as pl
from jax.experimental.pallas import tpu as pltpu
from jax.experimental.pallas import tpu_sc as plsc

_N = 10000
_E = 320000
_D_IN = 128
_HID = 128
_D_OUT = 64
_NE = 8
_COEF = 0.01
_NUM_LAYERS = 4

_NP = 10240
_ROWS_PER_TILE = _NP // 16
_BA = 1000
_GRID = _N // _BA
_EDGE_B = 128
_NW = 32
_K = 79
_EPAD = _NW * _K * _EDGE_B


def _prelude_body(x_ref, noise_ref, win_ref, bin_ref, wg_ref, wn_ref,
                  h_ref, logits_ref, gates_ref, imp_ref, load_ref):
    i = pl.program_id(0)
    x = x_ref[...]
    h = jnp.maximum(
        jnp.dot(x, win_ref[...], preferred_element_type=jnp.float32)
        + bin_ref[...], 0.0)
    h_ref[...] = h
    clean = jnp.dot(h, wg_ref[...], preferred_element_type=jnp.float32)
    noisy = jnp.dot(h, wn_ref[...], preferred_element_type=jnp.float32)
    sp = jnp.maximum(noisy, 0.0) + jnp.log1p(jnp.exp(-jnp.abs(noisy)))
    raw = clean + noise_ref[...] * (sp + 1e-2)
    logits_ref[...] = raw
    col = lax.broadcasted_iota(jnp.int32, raw.shape, 1)
    m1 = jnp.max(raw, axis=1, keepdims=True)
    i1 = jnp.min(jnp.where(raw == m1, col, _NE), axis=1, keepdims=True)
    masked = jnp.where(col == i1, -jnp.inf, raw)
    m2 = jnp.max(masked, axis=1, keepdims=True)
    i2 = jnp.min(jnp.where(masked == m2, col, _NE), axis=1, keepdims=True)
    e2 = jnp.exp(m2 - m1)
    denom = 1.0 + e2
    gates = (jnp.where(col == i1, 1.0 / denom, 0.0)
             + jnp.where(col == i2, e2 / denom, 0.0))
    gates_ref[...] = gates

    @pl.when(i == 0)
    def _():
        imp_ref[...] = jnp.zeros_like(imp_ref)
        load_ref[...] = jnp.zeros_like(load_ref)

    imp_ref[...] += jnp.sum(gates, axis=0, keepdims=True)
    load_ref[...] += jnp.sum(
        jnp.where(gates > 0.0, 1.0, 0.0), axis=0, keepdims=True)


def _prelude(x, noise, w_in, b_in2, w_gate, w_noise):
    return pl.pallas_call(
        _prelude_body,
        grid=(_GRID,),
        in_specs=[
            pl.BlockSpec((_BA, _D_IN), lambda i: (i, 0)),
            pl.BlockSpec((_BA, _NE), lambda i: (i, 0)),
            pl.BlockSpec((_D_IN, _HID), lambda i: (0, 0)),
            pl.BlockSpec((1, _HID), lambda i: (0, 0)),
            pl.BlockSpec((_HID, _NE), lambda i: (0, 0)),
            pl.BlockSpec((_HID, _NE), lambda i: (0, 0)),
        ],
        out_specs=[
            pl.BlockSpec((_BA, _HID), lambda i: (i, 0)),
            pl.BlockSpec((_BA, _NE), lambda i: (i, 0)),
            pl.BlockSpec((_BA, _NE), lambda i: (i, 0)),
            pl.BlockSpec((1, _NE), lambda i: (0, 0)),
            pl.BlockSpec((1, _NE), lambda i: (0, 0)),
        ],
        out_shape=[
            jax.ShapeDtypeStruct((_N, _HID), jnp.float32),
            jax.ShapeDtypeStruct((_N, _NE), jnp.float32),
            jax.ShapeDtypeStruct((_N, _NE), jnp.float32),
            jax.ShapeDtypeStruct((1, _NE), jnp.float32),
            jax.ShapeDtypeStruct((1, _NE), jnp.float32),
        ],
    )(x, noise, w_in, b_in2, w_gate, w_noise)


@functools.cache
def _make_sc_aggregate():
    mesh = plsc.VectorSubcoreMesh(core_axis_name="c", subcore_axis_name="s")

    @functools.partial(
        pl.kernel,
        mesh=mesh,
        out_type=[
            jax.ShapeDtypeStruct((2, _NP, _HID), jnp.float32),
            jax.ShapeDtypeStruct((2, _NP), jnp.float32),
        ],
        scratch_types=[
            pltpu.VMEM((_EDGE_B, _HID), jnp.float32),
            pltpu.VMEM((_K, _EDGE_B), jnp.int32),
            pltpu.VMEM((_K, _EDGE_B), jnp.int32),
            pltpu.VMEM((_EDGE_B,), jnp.float32),
            pltpu.VMEM((_ROWS_PER_TILE,), jnp.float32),
            pltpu.VMEM_SHARED((_NP, _HID), jnp.float32),
            pltpu.VMEM_SHARED((_NP,), jnp.float32),
            pltpu.SemaphoreType.DMA,
            pltpu.SemaphoreType.DMA,
        ],
    )
    def sc_aggregate(src_hbm, dst_hbm, h_hbm, agg_out, deg_out,
                     rows_a, src_v, dst_v, ones_v, dz_v,
                     agg_sh, deg_sh, g_a, d_s):
        c = lax.axis_index("c")
        s = lax.axis_index("s")
        wid = c * 16 + s
        base = s * _ROWS_PER_TILE

        def _zrow(r, _):
            def _zlane(j, _):
                rows_a[r, pl.ds(j * 16, 16)] = jnp.zeros((16,), jnp.float32)
                return 0
            return lax.fori_loop(0, _HID // 16, _zlane, 0)
        lax.fori_loop(0, _EDGE_B, _zrow, 0)

        def _fill(j, _):
            ones_v[pl.ds(j * 16, 16)] = jnp.full((16,), 1.0, jnp.float32)
            return 0
        lax.fori_loop(0, _EDGE_B // 16, _fill, 0)

        def _fill2(j, _):
            dz_v[pl.ds(j * 16, 16)] = jnp.zeros((16,), jnp.float32)
            return 0
        lax.fori_loop(0, _ROWS_PER_TILE // 16, _fill2, 0)

        for t in range(_ROWS_PER_TILE // _EDGE_B):
            pltpu.sync_copy(rows_a, agg_sh.at[pl.ds(base + t * _EDGE_B,
                                                    _EDGE_B)])
        pltpu.sync_copy(dz_v, deg_sh.at[pl.ds(base, _ROWS_PER_TILE)])
        plsc.subcore_barrier()

        pltpu.sync_copy(src_hbm.at[wid], src_v)
        pltpu.sync_copy(dst_hbm.at[wid], dst_v)
        pltpu.async_copy(h_hbm.at[src_v.at[0]], rows_a, g_a)

        def _edge_row(j, _):
            pltpu.make_async_copy(h_hbm.at[src_v.at[j]], rows_a, g_a).wait()
            pltpu.sync_copy(rows_a, agg_sh.at[dst_v.at[j]], add=True)

            @pl.when(j < _K - 1)
            def _():
                pltpu.async_copy(h_hbm.at[src_v.at[j + 1]], rows_a, g_a)

            @pl.when(j > 0)
            def _():
                pltpu.make_async_copy(ones_v, deg_sh.at[dst_v.at[j]],
                                      d_s).wait()

            pltpu.async_copy(ones_v, deg_sh.at[dst_v.at[j]], d_s, add=True)
            return 0
        lax.fori_loop(0, _K, _edge_row, 0)
        pltpu.make_async_copy(ones_v, deg_sh.at[dst_v.at[0]], d_s).wait()
        plsc.subcore_barrier()

        pltpu.sync_copy(agg_sh.at[pl.ds(base, _ROWS_PER_TILE)],
                        agg_out.at[c, pl.ds(base, _ROWS_PER_TILE)])
        pltpu.sync_copy(deg_sh.at[pl.ds(base, _ROWS_PER_TILE)],
                        deg_out.at[c, pl.ds(base, _ROWS_PER_TILE)])

    return sc_aggregate


def _expert_body(aggp_ref, degp_ref, gates_ref, we_ref, be_ref, wo_ref,
                 bo_ref, imp_ref, load_ref, out_ref, lb_ref):
    i = pl.program_id(0)
    agg = aggp_ref[0] + aggp_ref[1]
    deg = degp_ref[0] + degp_ref[1]
    agg = agg / jnp.maximum(deg, 1.0)
    acc = jnp.zeros((_BA, _HID), jnp.float32)
    g = gates_ref[...]
    for e in range(_NE):
        eo = jnp.maximum(
            jnp.dot(agg, we_ref[e], preferred_element_type=jnp.float32)
            + be_ref[...][e:e + 1, :], 0.0)
        acc += g[:, e:e + 1] * eo
    out_ref[...] = (jnp.dot(acc, wo_ref[...],
                            preferred_element_type=jnp.float32)
                    + bo_ref[...])

    @pl.when(i == 0)
    def _():
        def cv2(v):
            m = jnp.sum(v) / _NE
            var = jnp.sum((v - m) ** 2) / (_NE - 1)
            return var / (m * m + 1e-10)
        lb = _COEF * (cv2(imp_ref[...]) + cv2(load_ref[...]))
        lb = lb / math.ceil((_NUM_LAYERS - 2) / 2)
        lb_ref[...] = jnp.full((1, 1), lb, jnp.float32)


def _experts(aggp, degp_r, gates, w_expert, b_expert, w_out, b_out2,
             imp, load):
    return pl.pallas_call(
        _expert_body,
        grid=(_GRID,),
        in_specs=[
            pl.BlockSpec((2, _BA, _HID), lambda i: (0, i, 0)),
            pl.BlockSpec((2, _BA, 1), lambda i: (0, i, 0)),
            pl.BlockSpec((_BA, _NE), lambda i: (i, 0)),
            pl.BlockSpec((_NE, _HID, _HID), lambda i: (0, 0, 0)),
            pl.BlockSpec((_NE, _HID), lambda i: (0, 0)),
            pl.BlockSpec((_HID, _D_OUT), lambda i: (0, 0)),
            pl.BlockSpec((1, _D_OUT), lambda i: (0, 0)),
            pl.BlockSpec((1, _NE), lambda i: (0, 0)),
            pl.BlockSpec((1, _NE), lambda i: (0, 0)),
        ],
        out_specs=[
            pl.BlockSpec((_BA, _D_OUT), lambda i: (i, 0)),
            pl.BlockSpec((1, 1), lambda i: (0, 0)),
        ],
        out_shape=[
            jax.ShapeDtypeStruct((_N, _D_OUT), jnp.float32),
            jax.ShapeDtypeStruct((1, 1), jnp.float32),
        ],
    )(aggp, degp_r, gates, w_expert, b_expert, w_out, b_out2, imp, load)


def kernel(x, edge_index, noise, W_in, b_in, w_gate, w_noise, W_expert,
           b_expert, W_out, b_out):
    h, raw, gates, imp, load = _prelude(
        x, noise, W_in, b_in.reshape(1, _HID), w_gate, w_noise)

    src = edge_index[0]
    dst = edge_index[1]
    npad = _EPAD - _E
    src_p = jnp.concatenate(
        [src, jnp.zeros((npad,), jnp.int32)]).reshape(_NW, _K, _EDGE_B)
    pad_dst = _N + (jnp.arange(npad, dtype=jnp.int32) % (_NP - _N))
    dst_p = jnp.concatenate([dst, pad_dst]).reshape(_NW, _K, _EDGE_B)

    aggp, degp = _make_sc_aggregate()(src_p, dst_p, h)

    out, lb = _experts(aggp, degp.reshape(2, _NP, 1), gates, W_expert,
                       b_expert, W_out, b_out.reshape(1, _D_OUT),
                       imp, load)
    return out, lb.reshape(()), raw
